# baseline (speedup 1.0000x reference)
import functools

import jax
import jax.numpy as jnp
from jax import lax
from jax.experimental import pallas as pl
from jax.experimental.pallas import tpu as pltpu
from jax.experimental.pallas import tpu_sc as plsc

NUM_FEATURES = 4096
BATCH = 16384

_info = plsc.get_sparse_core_info()
_NC, _NS, _L = _info.num_cores, _info.num_subcores, _info.num_lanes
_NW = _NC * _NS
_ROWS_PER_W = BATCH // _NW
_R = 8
_CHUNKS = _ROWS_PER_W // _R
_KSTEPS = NUM_FEATURES // _L


def _permute_sc(x, perm32):
    mesh = plsc.VectorSubcoreMesh(core_axis_name="c", subcore_axis_name="s")

    @functools.partial(
        pl.kernel,
        mesh=mesh,
        out_type=jax.ShapeDtypeStruct((BATCH * NUM_FEATURES,), jnp.float32),
        compiler_params=pltpu.CompilerParams(needs_layout_passes=False),
        scratch_types=[
            pltpu.VMEM((NUM_FEATURES,), jnp.int32),
            pltpu.VMEM((_R * NUM_FEATURES,), jnp.float32),
            pltpu.VMEM((_R * NUM_FEATURES,), jnp.float32),
        ],
    )
    def permute(x_hbm, perm_hbm, out_hbm, perm_v, in_v, out_v):
        wid = lax.axis_index("s") * _NC + lax.axis_index("c")
        pltpu.sync_copy(perm_hbm, perm_v)
        base0 = wid * _ROWS_PER_W

        def chunk_body(c, carry):
            base = (base0 + c * _R) * NUM_FEATURES
            pltpu.sync_copy(x_hbm.at[pl.ds(base, _R * NUM_FEATURES)], in_v)

            def kbody(kk, inner):
                idx = perm_v[pl.ds(kk * _L, _L)]
                for r in range(_R):
                    val = plsc.load_gather(in_v, [idx + r * NUM_FEATURES])
                    out_v[pl.ds(r * NUM_FEATURES + kk * _L, _L)] = val
                return inner

            lax.fori_loop(0, _KSTEPS, kbody, 0)
            pltpu.sync_copy(out_v, out_hbm.at[pl.ds(base, _R * NUM_FEATURES)])
            return carry

        lax.fori_loop(0, _CHUNKS, chunk_body, 0)

    flat = permute(x.reshape(-1), perm32)
    return flat.reshape(BATCH, NUM_FEATURES)


def kernel(x, perm, inv_perm):
    del inv_perm
    return _permute_sc(x, perm.astype(jnp.int32))


# speedup vs baseline: 2.0278x; 2.0278x over previous
import functools

import jax
import jax.numpy as jnp
from jax import lax
from jax.experimental import pallas as pl
from jax.experimental.pallas import tpu as pltpu
from jax.experimental.pallas import tpu_sc as plsc

NUM_FEATURES = 4096
BATCH = 16384

_info = plsc.get_sparse_core_info()
_NC, _NS, _L = _info.num_cores, _info.num_subcores, _info.num_lanes
_NW = _NC * _NS
_ROWS_PER_W = BATCH // _NW
_R = 4
_CHUNK = _R * NUM_FEATURES
_CHUNKS = _ROWS_PER_W // _R
_KSTEPS = NUM_FEATURES // _L


def _permute_sc(x, perm32):
    mesh = plsc.VectorSubcoreMesh(core_axis_name="c", subcore_axis_name="s")

    @functools.partial(
        pl.kernel,
        mesh=mesh,
        out_type=jax.ShapeDtypeStruct((BATCH * NUM_FEATURES,), jnp.float32),
        compiler_params=pltpu.CompilerParams(needs_layout_passes=False),
        scratch_types=[
            pltpu.VMEM((NUM_FEATURES,), jnp.int32),
            pltpu.VMEM((_CHUNK,), jnp.float32),
            pltpu.VMEM((_CHUNK,), jnp.float32),
            pltpu.VMEM((_CHUNK,), jnp.float32),
            pltpu.VMEM((_CHUNK,), jnp.float32),
            pltpu.SemaphoreType.DMA,
            pltpu.SemaphoreType.DMA,
            pltpu.SemaphoreType.DMA,
            pltpu.SemaphoreType.DMA,
        ],
    )
    def permute(x_hbm, perm_hbm, out_hbm, perm_v, in0, in1, out0, out1,
                isem0, isem1, osem0, osem1):
        wid = lax.axis_index("s") * _NC + lax.axis_index("c")
        pltpu.sync_copy(perm_hbm, perm_v)
        base0 = wid * _ROWS_PER_W * NUM_FEATURES
        ins, outs = (in0, in1), (out0, out1)
        isems, osems = (isem0, isem1), (osem0, osem1)

        def in_copy(c, b):
            return pltpu.make_async_copy(
                x_hbm.at[pl.ds(base0 + c * _CHUNK, _CHUNK)], ins[b], isems[b])

        def out_copy(c, b):
            return pltpu.make_async_copy(
                outs[b], out_hbm.at[pl.ds(base0 + c * _CHUNK, _CHUNK)], osems[b])

        in_copy(0, 0).start()
        in_copy(1, 1).start()

        @pl.loop(0, _CHUNKS, step=2)
        def chunk_loop(g):
            for b in range(2):
                c = g + b
                in_copy(c, b).wait()

                @pl.when(c >= 2)
                def _():
                    out_copy(c - 2, b).wait()

                inb, outb = ins[b], outs[b]

                @plsc.parallel_loop(0, _KSTEPS, unroll=8)
                def kbody(kk):
                    idx = perm_v[pl.ds(kk * _L, _L)]
                    for r in range(_R):
                        val = plsc.load_gather(inb, [idx + r * NUM_FEATURES])
                        outb[pl.ds(r * NUM_FEATURES + kk * _L, _L)] = val

                out_copy(c, b).start()

                @pl.when(c + 2 < _CHUNKS)
                def _():
                    in_copy(c + 2, b).start()

        out_copy(_CHUNKS - 2, 0).wait()
        out_copy(_CHUNKS - 1, 1).wait()

    flat = permute(x.reshape(-1), perm32)
    return flat.reshape(BATCH, NUM_FEATURES)


def kernel(x, perm, inv_perm):
    del inv_perm
    return _permute_sc(x, perm.astype(jnp.int32))
